# baseline (speedup 1.0000x reference)
import functools

import jax
import jax.numpy as jnp
from jax import lax
from jax.experimental import pallas as pl
from jax.experimental.pallas import tpu as pltpu
from jax.experimental.pallas import tpu_sc as plsc

N_NODES = 10000
N_EDGES = 320000
F = 128

NC = 2
NS = 16
NW = NC * NS

CHUNK = 64
CH_PER_TILE = 160
CH_HALF = 40
NBUF = 4
DEG_CHUNK = 128
E_PAD = NW * CH_PER_TILE * CHUNK
N_PAD = 10240
ROWS_PER_TILE = N_PAD // NS
DUMMY_COL = N_NODES

_mesh = plsc.VectorSubcoreMesh(
    core_axis_name="c", subcore_axis_name="s", num_cores=NC, num_subcores=NS)


def _deg_body(col2d_hbm, zeros1_hbm, deg_hbm, cidx_v, ones_v, deg_acc):
    cid = lax.axis_index("c")
    sid = lax.axis_index("s")

    @pl.when(cid == 0)
    def _():
        pltpu.sync_copy(zeros1_hbm, deg_acc.at[pl.ds(sid * ROWS_PER_TILE,
                                                     ROWS_PER_TILE)])
        for j in range(DEG_CHUNK // 16):
            ones_v[pl.ds(j * 16, 16)] = jnp.ones((16,), jnp.float32)
        plsc.subcore_barrier()

        rows = (E_PAD // DEG_CHUNK) // NS
        pltpu.sync_copy(col2d_hbm.at[pl.ds(sid * rows, rows)], cidx_v)

        def body(k, carry):
            pltpu.sync_copy(ones_v, deg_acc.at[cidx_v.at[k]], add=True)
            return carry
        lax.fori_loop(0, rows, body, 0)
        plsc.subcore_barrier()

        base = sid * ROWS_PER_TILE
        pltpu.sync_copy(deg_acc.at[pl.ds(base, ROWS_PER_TILE)],
                        deg_hbm.at[pl.ds(base, ROWS_PER_TILE)])


_deg = functools.partial(
    pl.kernel, _deg_body, mesh=_mesh,
    out_type=jax.ShapeDtypeStruct((N_PAD,), jnp.float32),
    scratch_types=[
        pltpu.VMEM(((E_PAD // DEG_CHUNK) // NS, DEG_CHUNK), jnp.int32),
        pltpu.VMEM((DEG_CHUNK,), jnp.float32),
        pltpu.VMEM_SHARED((N_PAD,), jnp.float32),
    ],
)()


def _spmm_body(row2d_hbm, col2d_hbm, hp_hbm, zeros2_hbm, part_hbm,
               ridx_v, cidx_v, r0, r1, r2, r3, g0, g1, g2, g3,
               s0, s1, s2, s3, acc):
    cid = lax.axis_index("c")
    sid = lax.axis_index("s")
    wid = cid * NS + sid
    bufs, gsem, ssem = [r0, r1, r2, r3], [g0, g1, g2, g3], [s0, s1, s2, s3]

    def gstart(c, j):
        pltpu.async_copy(hp_hbm.at[ridx_v.at[c]], bufs[j], gsem[j])

    def gwait(c, j):
        pltpu.make_async_copy(hp_hbm.at[ridx_v.at[c]], bufs[j], gsem[j]).wait()

    def sstart(c, j):
        pltpu.make_async_copy(bufs[j], acc.at[cidx_v.at[c]],
                              ssem[j]).start(add=True)

    def swait(c, j):
        pltpu.make_async_copy(bufs[j], acc.at[cidx_v.at[c]], ssem[j]).wait()

    pltpu.sync_copy(zeros2_hbm, acc.at[pl.ds(sid * ROWS_PER_TILE,
                                             ROWS_PER_TILE)])
    plsc.subcore_barrier()

    def body(kk, carry):
        for j in range(NBUF):
            c = kk * NBUF + j
            gwait(c, j)
            sstart(c, j)
            swait(c - 1, (j + 3) % NBUF)
            gstart(c + 3, (j + 3) % NBUF)
        return carry

    for h in range(CH_PER_TILE // CH_HALF):
        base = wid * CH_PER_TILE + h * CH_HALF
        pltpu.sync_copy(row2d_hbm.at[pl.ds(base, CH_HALF)], ridx_v)
        pltpu.sync_copy(col2d_hbm.at[pl.ds(base, CH_HALF)], cidx_v)
        gstart(0, 0)
        gstart(1, 1)
        gstart(2, 2)
        gwait(0, 0); sstart(0, 0); gstart(3, 3)
        gwait(1, 1); sstart(1, 1); swait(0, 0); gstart(4, 0)
        gwait(2, 2); sstart(2, 2); swait(1, 1); gstart(5, 1)
        gwait(3, 3); sstart(3, 3); swait(2, 2); gstart(6, 2)
        lax.fori_loop(1, CH_HALF // NBUF - 1, body, 0)
        c0 = CH_HALF - NBUF
        gwait(c0, 0); sstart(c0, 0); swait(c0 - 1, 3); gstart(c0 + 3, 3)
        gwait(c0 + 1, 1); sstart(c0 + 1, 1); swait(c0, 0)
        gwait(c0 + 2, 2); sstart(c0 + 2, 2); swait(c0 + 1, 1)
        gwait(c0 + 3, 3); sstart(c0 + 3, 3); swait(c0 + 2, 2)
        swait(c0 + 3, 3)
    plsc.subcore_barrier()

    base = sid * ROWS_PER_TILE
    pltpu.sync_copy(acc.at[pl.ds(base, ROWS_PER_TILE)],
                    part_hbm.at[cid, pl.ds(base, ROWS_PER_TILE)])


_spmm = functools.partial(
    pl.kernel, _spmm_body, mesh=_mesh,
    out_type=jax.ShapeDtypeStruct((NC, N_PAD, F), jnp.float32),
    scratch_types=[
        pltpu.VMEM((CH_HALF, CHUNK), jnp.int32),
        pltpu.VMEM((CH_HALF, CHUNK), jnp.int32),
        pltpu.VMEM((CHUNK, F), jnp.float32),
        pltpu.VMEM((CHUNK, F), jnp.float32),
        pltpu.VMEM((CHUNK, F), jnp.float32),
        pltpu.VMEM((CHUNK, F), jnp.float32),
        pltpu.SemaphoreType.DMA,
        pltpu.SemaphoreType.DMA,
        pltpu.SemaphoreType.DMA,
        pltpu.SemaphoreType.DMA,
        pltpu.SemaphoreType.DMA,
        pltpu.SemaphoreType.DMA,
        pltpu.SemaphoreType.DMA,
        pltpu.SemaphoreType.DMA,
        pltpu.VMEM_SHARED((N_PAD, F), jnp.float32),
    ],
)()


def _matmul_body(x_ref, w_ref, deg_ref, o_ref):
    isd = lax.rsqrt(jnp.maximum(deg_ref[...], jnp.float32(1.0)))
    o_ref[...] = jnp.dot(x_ref[...], w_ref[...],
                         preferred_element_type=jnp.float32) * isd


def _matmul_scaled(x, w, deg2d):
    blk = 1000
    return pl.pallas_call(
        _matmul_body,
        grid=(N_NODES // blk,),
        in_specs=[
            pl.BlockSpec((blk, F), lambda i: (i, 0)),
            pl.BlockSpec((F, F), lambda i: (0, 0)),
            pl.BlockSpec((blk, 1), lambda i: (i, 0)),
        ],
        out_specs=pl.BlockSpec((blk, F), lambda i: (i, 0)),
        out_shape=jax.ShapeDtypeStruct((N_NODES, F), jnp.float32),
    )(x, w, deg2d)


def _combine_body(p_ref, deg_ref, b_ref, o_ref):
    p = p_ref[...]
    isd = lax.rsqrt(jnp.maximum(deg_ref[...], jnp.float32(1.0)))
    o_ref[...] = (p[0] + p[1]) * isd + b_ref[...]


def _combine(partials, deg2d, b2d):
    blk = 1000
    return pl.pallas_call(
        _combine_body,
        grid=(N_NODES // blk,),
        in_specs=[
            pl.BlockSpec((NC, blk, F), lambda i: (0, i, 0)),
            pl.BlockSpec((blk, 1), lambda i: (i, 0)),
            pl.BlockSpec((1, F), lambda i: (0, 0)),
        ],
        out_specs=pl.BlockSpec((blk, F), lambda i: (i, 0)),
        out_shape=jax.ShapeDtypeStruct((N_NODES, F), jnp.float32),
    )(partials, deg2d, b2d)


def kernel(input_feature, edge_index, W, b):
    row = edge_index[0]
    col = edge_index[1]
    pad_t = (E_PAD - N_EDGES) // NW
    real_t = N_EDGES // NW
    dummy_cols = jnp.broadcast_to(
        DUMMY_COL + jnp.arange(pad_t, dtype=jnp.int32), (NW, pad_t))
    row_pad = jnp.concatenate(
        [row.reshape(NW, real_t),
         jnp.zeros((NW, pad_t), jnp.int32)], axis=1)
    col_pad = jnp.concatenate(
        [col.reshape(NW, real_t), dummy_cols], axis=1)
    row2d = row_pad.reshape(E_PAD // CHUNK, CHUNK)
    col2d = col_pad.reshape(E_PAD // CHUNK, CHUNK)
    col2d_deg = col_pad.reshape(E_PAD // DEG_CHUNK, DEG_CHUNK)
    zeros1 = jnp.zeros((ROWS_PER_TILE,), jnp.float32)
    zeros2 = jnp.zeros((ROWS_PER_TILE, F), jnp.float32)

    deg_pad = _deg(col2d_deg, zeros1)
    deg2d = deg_pad[:N_NODES].reshape(N_NODES, 1)
    hp = _matmul_scaled(input_feature, W, deg2d)
    partials = _spmm(row2d, col2d, hp, zeros2)
    return _combine(partials, deg2d, b.reshape(1, F))


# speedup vs baseline: 1.0256x; 1.0256x over previous
import functools

import jax
import jax.numpy as jnp
from jax import lax
from jax.experimental import pallas as pl
from jax.experimental.pallas import tpu as pltpu
from jax.experimental.pallas import tpu_sc as plsc

N_NODES = 10000
N_EDGES = 320000
F = 128

NC = 2
NS = 16
NW = NC * NS

CHUNK = 64
CH_PER_TILE = 160
CH_HALF = 40
NBUF = 4
DEG_CHUNK = 128
E_PAD = NW * CH_PER_TILE * CHUNK
N_PAD = 10240
ROWS_PER_TILE = N_PAD // NS
DUMMY_COL = N_NODES

_mesh = plsc.VectorSubcoreMesh(
    core_axis_name="c", subcore_axis_name="s", num_cores=NC, num_subcores=NS)


def _deg_body(col2d_hbm, deg_hbm, cidx_v, ones_v, zrow_v, deg_acc):
    cid = lax.axis_index("c")
    sid = lax.axis_index("s")

    for j in range(ROWS_PER_TILE // 16):
        zrow_v[pl.ds(j * 16, 16)] = jnp.zeros((16,), jnp.float32)
    pltpu.sync_copy(zrow_v, deg_acc.at[pl.ds(sid * ROWS_PER_TILE,
                                             ROWS_PER_TILE)])
    for j in range(DEG_CHUNK // 16):
        ones_v[pl.ds(j * 16, 16)] = jnp.ones((16,), jnp.float32)
    plsc.subcore_barrier()

    rows = (E_PAD // DEG_CHUNK) // NW
    pltpu.sync_copy(
        col2d_hbm.at[pl.ds((cid * NS + sid) * rows, rows)], cidx_v)

    def body(k, carry):
        pltpu.sync_copy(ones_v, deg_acc.at[cidx_v.at[k]], add=True)
        return carry
    lax.fori_loop(0, rows, body, 0)
    plsc.subcore_barrier()

    base = sid * ROWS_PER_TILE
    pltpu.sync_copy(deg_acc.at[pl.ds(base, ROWS_PER_TILE)],
                    deg_hbm.at[cid, pl.ds(base, ROWS_PER_TILE)])


_deg = functools.partial(
    pl.kernel, _deg_body, mesh=_mesh,
    out_type=jax.ShapeDtypeStruct((NC, N_PAD), jnp.float32),
    scratch_types=[
        pltpu.VMEM(((E_PAD // DEG_CHUNK) // NW, DEG_CHUNK), jnp.int32),
        pltpu.VMEM((DEG_CHUNK,), jnp.float32),
        pltpu.VMEM((ROWS_PER_TILE,), jnp.float32),
        pltpu.VMEM_SHARED((N_PAD,), jnp.float32),
    ],
)()


def _spmm_body(row2d_hbm, col2d_hbm, hp_hbm, part_hbm,
               ridx_v, cidx_v, r0, r1, r2, r3, g0, g1, g2, g3,
               s0, s1, s2, s3, acc):
    cid = lax.axis_index("c")
    sid = lax.axis_index("s")
    wid = cid * NS + sid
    bufs, gsem, ssem = [r0, r1, r2, r3], [g0, g1, g2, g3], [s0, s1, s2, s3]

    def gstart(c, j):
        pltpu.async_copy(hp_hbm.at[ridx_v.at[c]], bufs[j], gsem[j])

    def gwait(c, j):
        pltpu.make_async_copy(hp_hbm.at[ridx_v.at[c]], bufs[j], gsem[j]).wait()

    def sstart(c, j):
        pltpu.make_async_copy(bufs[j], acc.at[cidx_v.at[c]],
                              ssem[j]).start(add=True)

    def swait(c, j):
        pltpu.make_async_copy(bufs[j], acc.at[cidx_v.at[c]], ssem[j]).wait()

    def zrow(i, carry):
        for j in range(F // 16):
            r0[i, pl.ds(j * 16, 16)] = jnp.zeros((16,), jnp.float32)
        return carry
    lax.fori_loop(0, CHUNK, zrow, 0)
    for q in range(ROWS_PER_TILE // CHUNK):
        pltpu.sync_copy(r0, acc.at[pl.ds(sid * ROWS_PER_TILE + q * CHUNK,
                                         CHUNK)])
    plsc.subcore_barrier()

    def body(kk, carry):
        for j in range(NBUF):
            c = kk * NBUF + j
            gwait(c, j)
            sstart(c, j)
            swait(c - 1, (j + 3) % NBUF)
            gstart(c + 3, (j + 3) % NBUF)
        return carry

    for h in range(CH_PER_TILE // CH_HALF):
        base = wid * CH_PER_TILE + h * CH_HALF
        pltpu.sync_copy(row2d_hbm.at[pl.ds(base, CH_HALF)], ridx_v)
        pltpu.sync_copy(col2d_hbm.at[pl.ds(base, CH_HALF)], cidx_v)
        gstart(0, 0)
        gstart(1, 1)
        gstart(2, 2)
        gwait(0, 0); sstart(0, 0); gstart(3, 3)
        gwait(1, 1); sstart(1, 1); swait(0, 0); gstart(4, 0)
        gwait(2, 2); sstart(2, 2); swait(1, 1); gstart(5, 1)
        gwait(3, 3); sstart(3, 3); swait(2, 2); gstart(6, 2)
        lax.fori_loop(1, CH_HALF // NBUF - 1, body, 0)
        c0 = CH_HALF - NBUF
        gwait(c0, 0); sstart(c0, 0); swait(c0 - 1, 3); gstart(c0 + 3, 3)
        gwait(c0 + 1, 1); sstart(c0 + 1, 1); swait(c0, 0)
        gwait(c0 + 2, 2); sstart(c0 + 2, 2); swait(c0 + 1, 1)
        gwait(c0 + 3, 3); sstart(c0 + 3, 3); swait(c0 + 2, 2)
        swait(c0 + 3, 3)
    plsc.subcore_barrier()

    base = sid * ROWS_PER_TILE
    pltpu.sync_copy(acc.at[pl.ds(base, ROWS_PER_TILE)],
                    part_hbm.at[cid, pl.ds(base, ROWS_PER_TILE)])


_spmm = functools.partial(
    pl.kernel, _spmm_body, mesh=_mesh,
    out_type=jax.ShapeDtypeStruct((NC, N_PAD, F), jnp.float32),
    scratch_types=[
        pltpu.VMEM((CH_HALF, CHUNK), jnp.int32),
        pltpu.VMEM((CH_HALF, CHUNK), jnp.int32),
        pltpu.VMEM((CHUNK, F), jnp.float32),
        pltpu.VMEM((CHUNK, F), jnp.float32),
        pltpu.VMEM((CHUNK, F), jnp.float32),
        pltpu.VMEM((CHUNK, F), jnp.float32),
        pltpu.SemaphoreType.DMA,
        pltpu.SemaphoreType.DMA,
        pltpu.SemaphoreType.DMA,
        pltpu.SemaphoreType.DMA,
        pltpu.SemaphoreType.DMA,
        pltpu.SemaphoreType.DMA,
        pltpu.SemaphoreType.DMA,
        pltpu.SemaphoreType.DMA,
        pltpu.VMEM_SHARED((N_PAD, F), jnp.float32),
    ],
)()


def _matmul_body(x_ref, w_ref, deg_ref, o_ref):
    d = deg_ref[0] + deg_ref[1]
    isd = lax.rsqrt(jnp.maximum(d, jnp.float32(1.0)))
    o_ref[...] = jnp.dot(x_ref[...], w_ref[...],
                         preferred_element_type=jnp.float32) * isd


def _matmul_scaled(x, w, deg3d):
    blk = 1000
    return pl.pallas_call(
        _matmul_body,
        grid=(N_NODES // blk,),
        in_specs=[
            pl.BlockSpec((blk, F), lambda i: (i, 0)),
            pl.BlockSpec((F, F), lambda i: (0, 0)),
            pl.BlockSpec((NC, blk, 1), lambda i: (0, i, 0)),
        ],
        out_specs=pl.BlockSpec((blk, F), lambda i: (i, 0)),
        out_shape=jax.ShapeDtypeStruct((N_NODES, F), jnp.float32),
    )(x, w, deg3d)


def _combine_body(p_ref, deg_ref, b_ref, o_ref):
    p = p_ref[...]
    d = deg_ref[0] + deg_ref[1]
    isd = lax.rsqrt(jnp.maximum(d, jnp.float32(1.0)))
    o_ref[...] = (p[0] + p[1]) * isd + b_ref[...]


def _combine(partials, deg3d, b2d):
    blk = 1000
    return pl.pallas_call(
        _combine_body,
        grid=(N_NODES // blk,),
        in_specs=[
            pl.BlockSpec((NC, blk, F), lambda i: (0, i, 0)),
            pl.BlockSpec((NC, blk, 1), lambda i: (0, i, 0)),
            pl.BlockSpec((1, F), lambda i: (0, 0)),
        ],
        out_specs=pl.BlockSpec((blk, F), lambda i: (i, 0)),
        out_shape=jax.ShapeDtypeStruct((N_NODES, F), jnp.float32),
    )(partials, deg3d, b2d)


def kernel(input_feature, edge_index, W, b):
    row = edge_index[0]
    col = edge_index[1]
    pad_t = (E_PAD - N_EDGES) // NW
    real_t = N_EDGES // NW
    dummy_cols = jnp.broadcast_to(
        DUMMY_COL + jnp.arange(pad_t, dtype=jnp.int32), (NW, pad_t))
    row_pad = jnp.concatenate(
        [row.reshape(NW, real_t),
         jnp.zeros((NW, pad_t), jnp.int32)], axis=1)
    col_pad = jnp.concatenate(
        [col.reshape(NW, real_t), dummy_cols], axis=1)
    row2d = row_pad.reshape(E_PAD // CHUNK, CHUNK)
    col2d = col_pad.reshape(E_PAD // CHUNK, CHUNK)
    col2d_deg = col_pad.reshape(E_PAD // DEG_CHUNK, DEG_CHUNK)

    deg_pad = _deg(col2d_deg)
    deg3d = deg_pad[:, :N_NODES].reshape(NC, N_NODES, 1)
    hp = _matmul_scaled(input_feature, W, deg3d)
    partials = _spmm(row2d, col2d, hp)
    return _combine(partials, deg3d, b.reshape(1, F))
